# baseline (device time: 34512 ns/iter reference)
import jax
import jax.numpy as jnp
from jax import lax
from jax.experimental import pallas as pl
from jax.experimental.pallas import tpu as pltpu

N_DEV = 4
B = 2
S_PER = 128
HQ = 4
DH = 64
BH = B * HQ
D_MODEL = 512
BLK = 64
SCALE = 0.125


def kernel(x, Wq, K_ext, V_ext, Wo):
    def body(x_ref, wq_ref, k_ref, v_ref, wo_ref, out_ref,
             kv_loc, q_send, q_recv, part_send, part_recv,
             den_send, den_recv,
             qs_sems, qr_sems, ps_sems, pr_sems, ds_sems, dr_sems):
        my = lax.axis_index("i")
        bf16 = jnp.bfloat16
        NT = (((1,), (1,)), ((), ()))
        NN = (((1,), (0,)), ((), ()))

        barrier_sem = pltpu.get_barrier_semaphore()
        for o in range(1, N_DEV):
            pl.semaphore_signal(
                barrier_sem, inc=1,
                device_id=(lax.rem(my + o, N_DEV),),
                device_id_type=pl.DeviceIdType.MESH,
            )

        for b in range(B):
            for h in range(HQ):
                kv_loc[0, b * HQ + h] = k_ref[b, :, h, :].astype(bf16)
                kv_loc[1, b * HQ + h] = v_ref[b, :, h, :].astype(bf16)

        q16 = []
        for b in range(B):
            qf = jax.lax.dot_general(
                x_ref[b].astype(bf16), wq_ref[...].astype(bf16), NN,
                preferred_element_type=jnp.float32,
            ) * SCALE
            qb = []
            for h in range(HQ):
                qh = qf[:, h * DH:(h + 1) * DH].astype(bf16)
                q_send[b * HQ + h] = qh
                qb.append(qh)
            q16.append(qb)

        pl.semaphore_wait(barrier_sem, N_DEV - 1)

        def q_rdma(o):
            return pltpu.make_async_remote_copy(
                src_ref=q_send,
                dst_ref=q_recv.at[o - 1],
                send_sem=qs_sems.at[o - 1],
                recv_sem=qr_sems.at[o - 1],
                device_id=(my - o,),
                device_id_type=pl.DeviceIdType.MESH,
            )

        for o in range(1, N_DEV):
            @pl.when(my - o >= 0)
            def _(o=o):
                q_rdma(o).start()

        ib = lax.broadcasted_iota(jnp.int32, (S_PER, S_PER), 0) // BLK
        jb = lax.broadcasted_iota(jnp.int32, (S_PER, S_PER), 1) // BLK
        tri = jb <= ib

        ctx_acc = [[None] * HQ for _ in range(B)]
        den_acc = [[None] * HQ for _ in range(B)]
        for b in range(B):
            for h in range(HQ):
                idx = b * HQ + h
                s = jax.lax.dot_general(
                    q16[b][h], kv_loc[0, idx], NT,
                    preferred_element_type=jnp.float32,
                )
                e = jnp.exp(jnp.where(tri, s, -1e9))
                ctx_acc[b][h] = jax.lax.dot_general(
                    e.astype(bf16), kv_loc[1, idx], NN,
                    preferred_element_type=jnp.float32,
                )
                den_acc[b][h] = jnp.sum(e, axis=1, keepdims=True)

        def part_rdmas(o):
            ctx_r = pltpu.make_async_remote_copy(
                src_ref=part_send.at[o - 1],
                dst_ref=part_recv.at[o - 1],
                send_sem=ps_sems.at[o - 1],
                recv_sem=pr_sems.at[o - 1],
                device_id=(my + o,),
                device_id_type=pl.DeviceIdType.MESH,
            )
            den_r = pltpu.make_async_remote_copy(
                src_ref=den_send.at[o - 1],
                dst_ref=den_recv.at[o - 1],
                send_sem=ds_sems.at[o - 1],
                recv_sem=dr_sems.at[o - 1],
                device_id=(my + o,),
                device_id_type=pl.DeviceIdType.MESH,
            )
            return ctx_r, den_r

        for o in range(1, N_DEV):
            @pl.when(my + o < N_DEV)
            def _(o=o):
                pltpu.make_async_remote_copy(
                    src_ref=q_send, dst_ref=q_recv.at[o - 1],
                    send_sem=qs_sems.at[o - 1], recv_sem=qr_sems.at[o - 1],
                    device_id=(my,), device_id_type=pl.DeviceIdType.MESH,
                ).wait_recv()
                for idx in range(BH):
                    s = jax.lax.dot_general(
                        q_recv[o - 1, idx], kv_loc[0, idx], NT,
                        preferred_element_type=jnp.float32,
                    )
                    e = jnp.exp(s)
                    part_send[o - 1, idx] = jax.lax.dot_general(
                        e.astype(bf16), kv_loc[1, idx], NN,
                        preferred_element_type=jnp.float32,
                    ).astype(bf16)
                    den_send[o - 1, idx] = jnp.sum(e, axis=1, keepdims=True)
                ctx_r, den_r = part_rdmas(o)
                ctx_r.start()
                den_r.start()

        for o in range(1, N_DEV):
            @pl.when(my - o >= 0)
            def _(o=o):
                ctx_r, den_r = part_rdmas(o)
                ctx_r.wait_recv()
                den_r.wait_recv()
            valid = my - o >= 0
            for b in range(B):
                for h in range(HQ):
                    idx = b * HQ + h
                    pc = jnp.where(
                        valid, part_recv[o - 1, idx].astype(jnp.float32), 0.0
                    )
                    pd = jnp.where(valid, den_recv[o - 1, idx], 0.0)
                    ctx_acc[b][h] = ctx_acc[b][h] + pc
                    den_acc[b][h] = den_acc[b][h] + pd

        wo16 = wo_ref[...].astype(bf16)
        for b in range(B):
            ctx = jnp.concatenate(
                [ctx_acc[b][h] / den_acc[b][h] for h in range(HQ)], axis=1
            ).astype(bf16)
            out_ref[b] = jax.lax.dot_general(
                ctx, wo16, NN,
                preferred_element_type=jnp.float32,
            )

        for o in range(1, N_DEV):
            @pl.when(my - o >= 0)
            def _(o=o):
                q_rdma(o).wait_send()
            @pl.when(my + o < N_DEV)
            def _(o=o):
                ctx_r, den_r = part_rdmas(o)
                ctx_r.wait_send()
                den_r.wait_send()

    return pl.pallas_call(
        body,
        out_shape=jax.ShapeDtypeStruct((B, S_PER, D_MODEL), jnp.float32),
        in_specs=[pl.BlockSpec(memory_space=pltpu.VMEM)] * 5,
        out_specs=pl.BlockSpec(memory_space=pltpu.VMEM),
        scratch_shapes=[
            pltpu.VMEM((2, BH, S_PER, DH), jnp.bfloat16),
            pltpu.VMEM((BH, S_PER, DH), jnp.bfloat16),
            pltpu.VMEM((N_DEV - 1, BH, S_PER, DH), jnp.bfloat16),
            pltpu.VMEM((N_DEV - 1, BH, S_PER, DH), jnp.bfloat16),
            pltpu.VMEM((N_DEV - 1, BH, S_PER, DH), jnp.bfloat16),
            pltpu.VMEM((N_DEV - 1, BH, S_PER, 1), jnp.float32),
            pltpu.VMEM((N_DEV - 1, BH, S_PER, 1), jnp.float32),
            pltpu.SemaphoreType.DMA((N_DEV - 1,)),
            pltpu.SemaphoreType.DMA((N_DEV - 1,)),
            pltpu.SemaphoreType.DMA((N_DEV - 1,)),
            pltpu.SemaphoreType.DMA((N_DEV - 1,)),
            pltpu.SemaphoreType.DMA((N_DEV - 1,)),
            pltpu.SemaphoreType.DMA((N_DEV - 1,)),
        ],
        compiler_params=pltpu.CompilerParams(collective_id=0),
    )(x, Wq, K_ext, V_ext, Wo)


# device time: 15757 ns/iter; 2.1903x vs baseline; 2.1903x over previous
import jax
import jax.numpy as jnp
from jax import lax
from jax.experimental import pallas as pl
from jax.experimental.pallas import tpu as pltpu

N_DEV = 4
B = 2
S_PER = 128
HQ = 4
DH = 64
D_QK = HQ * DH
D_MODEL = 512
BLK = 64
SCALE = 0.125


def kernel(x, Wq, K_ext, V_ext, Wo):
    def body(x_ref, wq_ref, k_ref, v_ref, wo_ref, out_ref,
             kv_all, send_sems, recv_sems):
        my = lax.axis_index("i")
        bf16 = jnp.bfloat16
        NT = (((1,), (1,)), ((), ()))
        NN = (((1,), (0,)), ((), ()))

        barrier_sem = pltpu.get_barrier_semaphore()
        for o in range(1, N_DEV):
            @pl.when(my - o >= 0)
            def _(o=o):
                pl.semaphore_signal(
                    barrier_sem, inc=1,
                    device_id=(my - o,),
                    device_id_type=pl.DeviceIdType.MESH,
                )

        for b in range(B):
            kv_all[my, 0, b] = k_ref[b].reshape(S_PER, D_QK).astype(bf16)
            kv_all[my, 1, b] = v_ref[b].reshape(S_PER, D_QK).astype(bf16)

        def pair_rdma(o):
            return pltpu.make_async_remote_copy(
                src_ref=kv_all.at[my],
                dst_ref=kv_all.at[my],
                send_sem=send_sems.at[o - 1],
                recv_sem=recv_sems.at[o - 1],
                device_id=(my + o,),
                device_id_type=pl.DeviceIdType.MESH,
            )

        pl.semaphore_wait(barrier_sem, N_DEV - 1 - my)
        for o in range(1, N_DEV):
            @pl.when(my + o < N_DEV)
            def _(o=o):
                pair_rdma(o).start()

        q16 = []
        for b in range(B):
            qf = jax.lax.dot_general(
                x_ref[b].astype(bf16), wq_ref[...].astype(bf16), NN,
                preferred_element_type=jnp.float32,
            ) * SCALE
            q16.append(qf.astype(bf16))

        ib = lax.broadcasted_iota(jnp.int32, (S_PER, S_PER), 0) // BLK
        jb = lax.broadcasted_iota(jnp.int32, (S_PER, S_PER), 1) // BLK
        tri = jb <= ib

        ctx_acc = [[None] * HQ for _ in range(B)]
        den_acc = [[None] * HQ for _ in range(B)]

        for o in range(N_DEV):
            if o > 0:
                @pl.when(my - o >= 0)
                def _(o=o):
                    pltpu.make_async_remote_copy(
                        src_ref=kv_all.at[0],
                        dst_ref=kv_all.at[0],
                        send_sem=send_sems.at[o - 1],
                        recv_sem=recv_sems.at[o - 1],
                        device_id=(my,),
                        device_id_type=pl.DeviceIdType.MESH,
                    ).wait_recv()
            slot = jnp.maximum(my - o, 0)
            valid = (my >= o).astype(jnp.float32)
            for b in range(B):
                k_full = kv_all[slot, 0, b]
                v_full = kv_all[slot, 1, b]
                for h in range(HQ):
                    sl = slice(h * DH, (h + 1) * DH)
                    s = jax.lax.dot_general(
                        q16[b][:, sl], k_full[:, sl], NT,
                        preferred_element_type=jnp.float32,
                    )
                    if o == 0:
                        s = jnp.where(tri, s, -1e9)
                    e = jnp.exp(s)
                    c = jax.lax.dot_general(
                        e.astype(bf16), v_full[:, sl], NN,
                        preferred_element_type=jnp.float32,
                    )
                    d = jnp.sum(e, axis=1, keepdims=True)
                    if o == 0:
                        ctx_acc[b][h] = c
                        den_acc[b][h] = d
                    else:
                        ctx_acc[b][h] = ctx_acc[b][h] + c * valid
                        den_acc[b][h] = den_acc[b][h] + d * valid

        wo16 = wo_ref[...].astype(bf16)
        for b in range(B):
            ctx = jnp.concatenate(
                [ctx_acc[b][h] / den_acc[b][h] for h in range(HQ)], axis=1
            ).astype(bf16)
            out_ref[b] = jax.lax.dot_general(
                ctx, wo16, NN,
                preferred_element_type=jnp.float32,
            )

        for o in range(1, N_DEV):
            @pl.when(my + o < N_DEV)
            def _(o=o):
                pair_rdma(o).wait_send()

    return pl.pallas_call(
        body,
        out_shape=jax.ShapeDtypeStruct((B, S_PER, D_MODEL), jnp.float32),
        in_specs=[pl.BlockSpec(memory_space=pltpu.VMEM)] * 5,
        out_specs=pl.BlockSpec(memory_space=pltpu.VMEM),
        scratch_shapes=[
            pltpu.VMEM((N_DEV, 2, B, S_PER, D_QK), jnp.bfloat16),
            pltpu.SemaphoreType.DMA((N_DEV - 1,)),
            pltpu.SemaphoreType.DMA((N_DEV - 1,)),
        ],
        compiler_params=pltpu.CompilerParams(collective_id=0),
    )(x, Wq, K_ext, V_ext, Wo)


# device time: 15234 ns/iter; 2.2655x vs baseline; 1.0343x over previous
import jax
import jax.numpy as jnp
from jax import lax
from jax.experimental import pallas as pl
from jax.experimental.pallas import tpu as pltpu

N_DEV = 4
B = 2
S_PER = 128
HQ = 4
DH = 64
D_QK = HQ * DH
D_MODEL = 512
BLK = 64
SCALE = 0.125


def kernel(x, Wq, K_ext, V_ext, Wo):
    K2 = K_ext.reshape(B, S_PER, D_QK).astype(jnp.bfloat16)
    V2 = V_ext.reshape(B, S_PER, D_QK).astype(jnp.bfloat16)

    def body(x_ref, wq_ref, k_ref, v_ref, wo_ref, out_ref,
             kv_all, send_sems, recv_sems):
        my = lax.axis_index("i")
        bf16 = jnp.bfloat16
        NT = (((1,), (1,)), ((), ()))
        NN = (((1,), (0,)), ((), ()))

        barrier_sem = pltpu.get_barrier_semaphore()
        for o in range(1, N_DEV):
            @pl.when(my - o >= 0)
            def _(o=o):
                pl.semaphore_signal(
                    barrier_sem, inc=1,
                    device_id=(my - o,),
                    device_id_type=pl.DeviceIdType.MESH,
                )

        for b in range(B):
            kv_all[my, 0, b] = k_ref[b]
            kv_all[my, 1, b] = v_ref[b]

        def pair_rdma(o):
            return pltpu.make_async_remote_copy(
                src_ref=kv_all.at[my],
                dst_ref=kv_all.at[my],
                send_sem=send_sems.at[o - 1],
                recv_sem=recv_sems.at[o - 1],
                device_id=(my + o,),
                device_id_type=pl.DeviceIdType.MESH,
            )

        pl.semaphore_wait(barrier_sem, N_DEV - 1 - my)
        for o in range(1, N_DEV):
            @pl.when(my + o < N_DEV)
            def _(o=o):
                pair_rdma(o).start()

        q16 = []
        for b in range(B):
            qf = jax.lax.dot_general(
                x_ref[b].astype(bf16), wq_ref[...].astype(bf16), NN,
                preferred_element_type=jnp.float32,
            ) * SCALE
            q16.append(qf.astype(bf16))

        ib = lax.broadcasted_iota(jnp.int32, (S_PER, S_PER), 0) // BLK
        jb = lax.broadcasted_iota(jnp.int32, (S_PER, S_PER), 1) // BLK
        tri = jb <= ib

        ctx_acc = [[None] * HQ for _ in range(B)]
        den_acc = [[None] * HQ for _ in range(B)]

        for o in range(N_DEV):
            if o > 0:
                @pl.when(my - o >= 0)
                def _(o=o):
                    pltpu.make_async_remote_copy(
                        src_ref=kv_all.at[0],
                        dst_ref=kv_all.at[0],
                        send_sem=send_sems.at[o - 1],
                        recv_sem=recv_sems.at[o - 1],
                        device_id=(my,),
                        device_id_type=pl.DeviceIdType.MESH,
                    ).wait_recv()
            slot = jnp.maximum(my - o, 0)
            valid = (my >= o).astype(jnp.float32)
            for b in range(B):
                k_full = kv_all[slot, 0, b]
                v_full = kv_all[slot, 1, b]
                for h in range(HQ):
                    sl = slice(h * DH, (h + 1) * DH)
                    s = jax.lax.dot_general(
                        q16[b][:, sl], k_full[:, sl], NT,
                        preferred_element_type=jnp.float32,
                    )
                    if o == 0:
                        s = jnp.where(tri, s, -1e9)
                    e = jnp.exp(s)
                    c = jax.lax.dot_general(
                        e.astype(bf16), v_full[:, sl], NN,
                        preferred_element_type=jnp.float32,
                    )
                    d = jnp.sum(e, axis=1, keepdims=True)
                    if o == 0:
                        ctx_acc[b][h] = c
                        den_acc[b][h] = d
                    else:
                        ctx_acc[b][h] = ctx_acc[b][h] + c * valid
                        den_acc[b][h] = den_acc[b][h] + d * valid

        wo16 = wo_ref[...].astype(bf16)
        for b in range(B):
            ctx = jnp.concatenate(
                [ctx_acc[b][h] / den_acc[b][h] for h in range(HQ)], axis=1
            ).astype(bf16)
            out_ref[b] = jax.lax.dot_general(
                ctx, wo16, NN,
                preferred_element_type=jnp.float32,
            )

        for o in range(1, N_DEV):
            @pl.when(my + o < N_DEV)
            def _(o=o):
                pair_rdma(o).wait_send()

    return pl.pallas_call(
        body,
        out_shape=jax.ShapeDtypeStruct((B, S_PER, D_MODEL), jnp.float32),
        in_specs=[pl.BlockSpec(memory_space=pltpu.VMEM)] * 5,
        out_specs=pl.BlockSpec(memory_space=pltpu.VMEM),
        scratch_shapes=[
            pltpu.VMEM((N_DEV, 2, B, S_PER, D_QK), jnp.bfloat16),
            pltpu.SemaphoreType.DMA((N_DEV - 1,)),
            pltpu.SemaphoreType.DMA((N_DEV - 1,)),
        ],
        compiler_params=pltpu.CompilerParams(collective_id=0),
    )(x, Wq, K2, V2, Wo)
